# baseline (speedup 1.0000x reference)
import jax
import jax.numpy as jnp
from jax import lax
from jax.experimental import pallas as pl
from jax.experimental.pallas import tpu as pltpu

_BN = 2048
_RS = 4


def _fhs_body(surf_ref, emb_ref, o_ref):
    s = surf_ref[...]
    B, T = s.shape
    V = emb_ref.shape[0]
    oh = (s[:, :, None] == lax.broadcasted_iota(jnp.int32, (B, T, V), 2))
    counts = jnp.sum(oh.astype(jnp.float32), axis=1)
    o_ref[...] = jnp.dot(
        counts, emb_ref[...], preferred_element_type=jnp.float32) * (1.0 / T)


def _proj_body(fhs_ref, b_ref, *rest):
    w_refs = rest[:_RS]
    o_ref = rest[_RS]
    D = fhs_ref.shape[1]
    dr = D // _RS
    i = pl.program_id(0)
    acc = jnp.dot(fhs_ref[:, 0:dr], w_refs[0][...],
                  preferred_element_type=jnp.float32)
    for j in range(1, _RS):
        acc += jnp.dot(fhs_ref[:, j * dr:(j + 1) * dr], w_refs[j][...],
                       preferred_element_type=jnp.float32)
    bias = b_ref[pl.ds(i * _BN, _BN)]
    o_ref[:, 0, :] = acc + bias[None, :]


def kernel(surf, char_emb, root_codebook, suffix_W, suffix_b, suffix_codebook,
           ord_W, ord_b, ord_codebooks, out_W, out_b):
    B, T = surf.shape
    V, D = char_emb.shape
    _, N = out_W.shape
    nb = (N + _BN - 1) // _BN
    dr = D // _RS

    fhs = pl.pallas_call(
        _fhs_body,
        out_shape=jax.ShapeDtypeStruct((B, D), jnp.float32),
    )(surf, char_emb)

    w_specs = [
        pl.BlockSpec((dr, _BN), lambda i, j=j: (j, i)) for j in range(_RS)
    ]
    out3d = pl.pallas_call(
        _proj_body,
        grid=(nb,),
        in_specs=[
            pl.BlockSpec((B, D), lambda i: (0, 0)),
            pl.BlockSpec((N,), lambda i: (0,)),
        ] + w_specs,
        out_specs=pl.BlockSpec((B, 1, _BN), lambda i: (0, 0, i)),
        out_shape=jax.ShapeDtypeStruct((B, 1, N), jnp.float32),
        compiler_params=pltpu.CompilerParams(
            dimension_semantics=("parallel",)),
    )(fhs, out_b, *([out_W] * _RS))
    return out3d


# speedup vs baseline: 1.2135x; 1.2135x over previous
import jax
import jax.numpy as jnp
from jax import lax
from jax.experimental import pallas as pl
from jax.experimental.pallas import tpu as pltpu

_BN = 8192


def _fhs_body(surf_ref, emb_ref, o_ref):
    s = surf_ref[...]
    B, T = s.shape
    V = emb_ref.shape[0]
    oh = (s[:, :, None] == lax.broadcasted_iota(jnp.int32, (B, T, V), 2))
    counts = jnp.sum(oh.astype(jnp.float32), axis=1)
    o_ref[...] = jnp.dot(
        counts, emb_ref[...], preferred_element_type=jnp.float32) * (1.0 / T)


def _proj_body(fhs_ref, w_ref, b_ref, o_ref):
    o_ref[...] = (
        jnp.dot(fhs_ref[...], w_ref[...], preferred_element_type=jnp.float32)
        + b_ref[...])


def kernel(surf, char_emb, root_codebook, suffix_W, suffix_b, suffix_codebook,
           ord_W, ord_b, ord_codebooks, out_W, out_b):
    B, T = surf.shape
    V, D = char_emb.shape
    _, N = out_W.shape
    nb = (N + _BN - 1) // _BN
    b2d = out_b.reshape(1, N)

    fhs = pl.pallas_call(
        _fhs_body,
        out_shape=jax.ShapeDtypeStruct((B, D), jnp.float32),
    )(surf, char_emb)

    out2d = pl.pallas_call(
        _proj_body,
        grid=(nb,),
        in_specs=[
            pl.BlockSpec((B, D), lambda i: (0, 0)),
            pl.BlockSpec((D, _BN), lambda i: (0, i)),
            pl.BlockSpec((1, _BN), lambda i: (0, i)),
        ],
        out_specs=pl.BlockSpec((B, _BN), lambda i: (0, i)),
        out_shape=jax.ShapeDtypeStruct((B, N), jnp.float32),
        compiler_params=pltpu.CompilerParams(
            dimension_semantics=("parallel",)),
    )(fhs, out_W, b2d)
    return out2d[:, None, :]
